# baseline (speedup 1.0000x reference)
import functools

import jax
import jax.numpy as jnp
from jax import lax
from jax.experimental import pallas as pl
from jax.experimental.pallas import tpu as pltpu
from jax.experimental.pallas import tpu_sc as plsc

_J = 200
_I = 4096
_D = 32
_NW = 32
_CW = _I // _NW
_L = 16
_NB = 4

_mesh = plsc.VectorSubcoreMesh(core_axis_name="c", subcore_axis_name="s")


@functools.partial(
    pl.kernel,
    out_type=jax.ShapeDtypeStruct((_J, 4, _NW, 8, 128), jnp.float32),
    mesh=_mesh,
    scratch_types=[
        pltpu.VMEM((_J, _CW), jnp.int32),
        pltpu.VMEM((_NB, _CW, _D), jnp.float32),
        pltpu.VMEM((2, 4, 8, 128), jnp.float32),
    ]
    + [pltpu.SemaphoreType.DMA] * (_NB + 2),
    compiler_params=pltpu.CompilerParams(
        use_tc_tiling_on_sc=False, needs_layout_passes=False
    ),
)
def _embed_sc(idx_hbm, tab_hbm, out_hbm, idx_all, buf, tb, *sems):
    gsem = sems[:_NB]
    ssem = sems[_NB:]
    wid = lax.axis_index("s") * 2 + lax.axis_index("c")
    col0 = wid * _CW

    pltpu.sync_copy(idx_hbm.at[:, pl.ds(col0, _CW)], idx_all)

    def start_gather(j, b):
        pltpu.async_copy(tab_hbm.at[idx_all.at[j]], buf.at[b], gsem[b])

    def wait_gather(j, b):
        pltpu.make_async_copy(
            tab_hbm.at[idx_all.at[j]], buf.at[b], gsem[b]
        ).wait()

    def start_store(j, b2):
        for tr in range(4):
            pltpu.async_copy(tb.at[b2, tr], out_hbm.at[j, tr, wid], ssem[b2])

    def wait_store(j, b2):
        for tr in range(4):
            pltpu.make_async_copy(
                tb.at[b2, tr], out_hbm.at[j, tr, wid], ssem[b2]
            ).wait()

    for b in range(_NB):
        start_gather(b, b)

    iota = lax.iota(jnp.int32, _L)
    cvecs = [jnp.full((_L,), c, jnp.int32) for c in range(_D)]

    def step(j, b, b2):
        wait_gather(j, b)

        @pl.when(j >= 2)
        def _():
            wait_store(j - 2, b2)

        for g in range(_CW // _L):
            rows = iota + (g * _L)
            for c in range(_D):
                val = plsc.load_gather(buf.at[b], [rows, cvecs[c]])
                tb[b2, c >> 3, c & 7, pl.ds(g * _L, _L)] = val
        start_store(j, b2)

        @pl.when(j + _NB < _J)
        def _():
            start_gather(j + _NB, b)

    def group(gg, carry):
        for b in range(_NB):
            step(gg * _NB + b, b, b % 2)
        return carry

    lax.fori_loop(0, _J // _NB, group, 0)

    wait_store(_J - 2, 0)
    wait_store(_J - 1, 1)


def kernel(idx, embeddings):
    idx2 = jnp.transpose(idx)
    out5 = _embed_sc(idx2, embeddings)
    return out5.transpose(2, 4, 0, 1, 3).reshape(4096, 200, 32)


# speedup vs baseline: 1.3897x; 1.3897x over previous
import functools

import jax
import jax.numpy as jnp
from jax import lax
from jax.experimental import pallas as pl
from jax.experimental.pallas import tpu as pltpu
from jax.experimental.pallas import tpu_sc as plsc

_J = 200
_I = 4096
_D = 32
_NW = 32
_CW = _I // _NW
_L = 16
_NB = 4

_mesh = plsc.VectorSubcoreMesh(core_axis_name="c", subcore_axis_name="s")


@functools.partial(
    pl.kernel,
    out_type=jax.ShapeDtypeStruct((_J, 4, _NW, 8, 128), jnp.float32),
    mesh=_mesh,
    scratch_types=[
        pltpu.VMEM((_J, _CW), jnp.int32),
        pltpu.VMEM((_NB, _CW, _D), jnp.float32),
        pltpu.VMEM((2, 4, 8, 128), jnp.float32),
    ]
    + [pltpu.SemaphoreType.DMA] * (_NB + 2),
    compiler_params=pltpu.CompilerParams(
        use_tc_tiling_on_sc=False, needs_layout_passes=False
    ),
)
def _embed_sc(idx_hbm, tab_hbm, out_hbm, idx_all, buf, tb, *sems):
    gsem = sems[:_NB]
    ssem = sems[_NB:]
    wid = lax.axis_index("s") * 2 + lax.axis_index("c")
    col0 = wid * _CW

    pltpu.sync_copy(idx_hbm.at[:, pl.ds(col0, _CW)], idx_all)

    def start_gather(j, b):
        pltpu.async_copy(tab_hbm.at[idx_all.at[j]], buf.at[b], gsem[b])

    def wait_gather(j, b):
        pltpu.make_async_copy(
            tab_hbm.at[idx_all.at[j]], buf.at[b], gsem[b]
        ).wait()

    def start_store(j, b2):
        for tr in range(4):
            pltpu.async_copy(tb.at[b2, tr], out_hbm.at[j, tr, wid], ssem[b2])

    def wait_store(j, b2):
        for tr in range(4):
            pltpu.make_async_copy(
                tb.at[b2, tr], out_hbm.at[j, tr, wid], ssem[b2]
            ).wait()

    for b in range(_NB):
        start_gather(b, b)

    iota = lax.iota(jnp.int32, _L)
    cvecs = [jnp.full((_L,), c, jnp.int32) for c in range(_D)]

    def step(j, b, b2):
        wait_gather(j, b)

        @pl.when(j >= 2)
        def _():
            wait_store(j - 2, b2)

        @plsc.parallel_loop(0, (_CW // _L) * _D, unroll=8)
        def _(t):
            g = t >> 5
            c = t & 31
            rows = iota + g * _L
            cvec = jnp.zeros((_L,), jnp.int32) + c
            val = plsc.load_gather(buf.at[b], [rows, cvec])
            tb[b2, c >> 3, c & 7, pl.ds(g * _L, _L)] = val

        start_store(j, b2)

        @pl.when(j + _NB < _J)
        def _():
            start_gather(j + _NB, b)

    def group(gg, carry):
        for b in range(_NB):
            step(gg * _NB + b, b, b % 2)
        return carry

    lax.fori_loop(0, _J // _NB, group, 0)

    wait_store(_J - 2, 0)
    wait_store(_J - 1, 1)


def kernel(idx, embeddings):
    idx2 = jnp.transpose(idx)
    out5 = _embed_sc(idx2, embeddings)
    return out5.transpose(2, 4, 0, 1, 3).reshape(4096, 200, 32)
